# baseline (speedup 1.0000x reference)
import functools

import jax
import jax.numpy as jnp
from jax import lax
from jax.experimental import pallas as pl
from jax.experimental.pallas import tpu as pltpu
from jax.experimental.pallas import tpu_sc as plsc

N_ATOMS = 10000
N_BONDS = 320000
MAX_NB = 32
H = 128
BOND_FDIM = 144

NC = 2
NS = 16
NW = NC * NS
APW = 320
N_PAD = NW * APW
BPW = N_BONDS // NW
GA = 2
NGROUPS = APW // GA
CB = 40
NCHUNK = BPW // CB
RING_C = 5

_mesh = plsc.VectorSubcoreMesh(
    core_axis_name="c", subcore_axis_name="s", num_cores=NC, num_subcores=NS)


def _wid():
  return lax.axis_index("c") * NS + lax.axis_index("s")


RING = 8


def _segsum_body(src_hbm, idx_hbm, out_hbm, idx_v, bufs, out_v, sems):
  base = _wid() * APW
  pltpu.sync_copy(idx_hbm.at[pl.ds(base * MAX_NB, APW * MAX_NB)], idx_v)

  def start(g, slot):
    pltpu.async_copy(
        src_hbm.at[idx_v.at[pl.ds(g * (GA * MAX_NB), GA * MAX_NB)]],
        bufs.at[slot], sems.at[slot])

  for s in range(RING):
    start(s, s)

  def outer(gg, carry):
    for slot in range(RING):
      g = RING * gg + slot
      buf = bufs.at[slot]
      pltpu.make_async_copy(
          src_hbm.at[idx_v.at[pl.ds(g * (GA * MAX_NB), GA * MAX_NB)]], buf,
          sems.at[slot]).wait()
      for a in range(GA):
        def rbody(kb, accs, a=a):
          r0 = a * MAX_NB + 4 * kb
          news = []
          for j in range(8):
            sl = pl.ds(16 * j, 16)
            v01 = buf[r0, sl] + buf[r0 + 1, sl]
            v23 = buf[r0 + 2, sl] + buf[r0 + 3, sl]
            news.append(accs[j] + (v01 + v23))
          return tuple(news)
        accs = lax.fori_loop(
            0, MAX_NB // 4, rbody,
            tuple(jnp.zeros((16,), jnp.float32) for _ in range(8)))
        r = slot * GA + a
        for j in range(8):
          out_v[r, 16 * j:16 * (j + 1)] = accs[j]

      @pl.when(g + RING < NGROUPS)
      def _():
        start(g + RING, slot)
    pltpu.sync_copy(out_v,
                    out_hbm.at[pl.ds(base + RING * GA * gg, RING * GA)])
    return carry

  lax.fori_loop(0, NGROUPS // RING, outer, 0)


def _segsum(src, idx_flat):
  fn = pl.kernel(
      _segsum_body,
      out_type=jax.ShapeDtypeStruct((N_PAD, H), jnp.float32),
      mesh=_mesh,
      scratch_types=[
          pltpu.VMEM((APW * MAX_NB,), jnp.int32),
          pltpu.VMEM((RING, GA * MAX_NB, H), jnp.float32),
          pltpu.VMEM((RING * GA, H), jnp.float32),
          pltpu.SemaphoreType.DMA((RING,)),
      ],
  )
  return fn(src, idx_flat)


def _combine_body(amh_hbm, mh_hbm, b2a_hbm, b2revb_hbm, inp_hbm, out_hbm,
                  idx1_v, idx2_v, bufs, out_bufs, sems, *, with_inp):
  base = _wid() * BPW
  pltpu.sync_copy(b2a_hbm.at[pl.ds(base, BPW)], idx1_v)
  pltpu.sync_copy(b2revb_hbm.at[pl.ds(base, BPW)], idx2_v)

  def start(c, slot):
    bufA, bufB, bufI = bufs.at[3 * slot], bufs.at[3 * slot + 1], bufs.at[
        3 * slot + 2]
    pltpu.async_copy(amh_hbm.at[idx1_v.at[pl.ds(c * CB, CB)]], bufA,
                     sems.at[3 * slot])
    pltpu.async_copy(mh_hbm.at[idx2_v.at[pl.ds(c * CB, CB)]], bufB,
                     sems.at[3 * slot + 1])
    if with_inp:
      pltpu.async_copy(inp_hbm.at[pl.ds(base + c * CB, CB)], bufI,
                       sems.at[3 * slot + 2])

  for s in range(RING_C):
    start(s, s)

  def outer(cc, carry):
    for slot in range(RING_C):
      c = RING_C * cc + slot
      bufA, bufB, bufI = bufs.at[3 * slot], bufs.at[3 * slot + 1], bufs.at[
          3 * slot + 2]
      out_v = out_bufs.at[slot]
      pltpu.make_async_copy(amh_hbm.at[idx1_v.at[pl.ds(c * CB, CB)]], bufA,
                            sems.at[3 * slot]).wait()
      pltpu.make_async_copy(mh_hbm.at[idx2_v.at[pl.ds(c * CB, CB)]], bufB,
                            sems.at[3 * slot + 1]).wait()
      if with_inp:
        pltpu.make_async_copy(inp_hbm.at[pl.ds(base + c * CB, CB)], bufI,
                              sems.at[3 * slot + 2]).wait()

      def rbody(rb, carry2):
        for rr in range(4):
          r = 4 * rb + rr
          for j in range(8):
            sl = pl.ds(16 * j, 16)
            v = bufA[r, sl] - bufB[r, sl]
            if with_inp:
              v = jnp.maximum(v + bufI[r, sl], 0.0)
            out_v[r, sl] = v
        return carry2

      lax.fori_loop(0, CB // 4, rbody, 0)
      pltpu.sync_copy(out_v, out_hbm.at[pl.ds(base + c * CB, CB)])

      @pl.when(c + RING_C < NCHUNK)
      def _():
        start(c + RING_C, slot)
    return carry

  lax.fori_loop(0, NCHUNK // RING_C, outer, 0)


def _combine(amh, mh, b2a, b2revb, inp, with_inp):
  fn = pl.kernel(
      functools.partial(_combine_body, with_inp=with_inp),
      out_type=jax.ShapeDtypeStruct((N_BONDS, H), jnp.float32),
      mesh=_mesh,
      scratch_types=[
          pltpu.VMEM((BPW,), jnp.int32),
          pltpu.VMEM((BPW,), jnp.int32),
          pltpu.VMEM((3 * RING_C, CB, H), jnp.float32),
          pltpu.VMEM((RING_C, CB, H), jnp.float32),
          pltpu.SemaphoreType.DMA((3 * RING_C,)),
      ],
  )
  return fn(amh, mh, b2a, b2revb, inp)


BE = 6400


def _k0_body(xT_ref, wiT_ref, whT_ref, inp_ref, mh_ref):
  inp = lax.dot_general(xT_ref[...], wiT_ref[...], (((0,), (0,)), ((), ())),
                        preferred_element_type=jnp.float32)
  inp_ref[...] = inp
  mh_ref[...] = jnp.dot(
      jnp.maximum(inp, 0.0), whT_ref[...], preferred_element_type=jnp.float32)


def _k0(f_bonds_T, wiT, whT):
  grid = (N_BONDS // BE,)
  return pl.pallas_call(
      _k0_body,
      grid=grid,
      in_specs=[
          pl.BlockSpec((BOND_FDIM, BE), lambda i: (0, i)),
          pl.BlockSpec((BOND_FDIM, H), lambda i: (0, 0)),
          pl.BlockSpec((H, H), lambda i: (0, 0)),
      ],
      out_specs=[
          pl.BlockSpec((BE, H), lambda i: (i, 0)),
          pl.BlockSpec((BE, H), lambda i: (i, 0)),
      ],
      out_shape=[
          jax.ShapeDtypeStruct((N_BONDS, H), jnp.float32),
          jax.ShapeDtypeStruct((N_BONDS, H), jnp.float32),
      ],
  )(f_bonds_T, wiT, whT)


def _k1_body(inp_ref, g_ref, whT_ref, mh_ref):
  m = jnp.maximum(inp_ref[...] + g_ref[...], 0.0)
  mh_ref[...] = jnp.dot(m, whT_ref[...], preferred_element_type=jnp.float32)


def _k1(inp, g, whT):
  grid = (N_BONDS // BE,)
  return pl.pallas_call(
      _k1_body,
      grid=grid,
      in_specs=[
          pl.BlockSpec((BE, H), lambda i: (i, 0)),
          pl.BlockSpec((BE, H), lambda i: (i, 0)),
          pl.BlockSpec((H, H), lambda i: (0, 0)),
      ],
      out_specs=pl.BlockSpec((BE, H), lambda i: (i, 0)),
      out_shape=jax.ShapeDtypeStruct((N_BONDS, H), jnp.float32),
  )(inp, g, whT)


BA = 2000
NMOL_PAD = 128


def _k3_body(fa_ref, am_ref, mol_ref, woaT_ref, wohT_ref, bo_ref, out_ref,
             acc_ref, cnt_ref):
  i = pl.program_id(0)

  @pl.when(i == 0)
  def _():
    acc_ref[...] = jnp.zeros_like(acc_ref)
    cnt_ref[...] = jnp.zeros_like(cnt_ref)

  hid = jnp.dot(fa_ref[...], woaT_ref[...], preferred_element_type=jnp.float32)
  hid = hid + jnp.dot(
      am_ref[...], wohT_ref[...], preferred_element_type=jnp.float32)
  hid = jnp.maximum(hid + bo_ref[...], 0.0)
  mol = mol_ref[0, 0, :]
  onehot = (mol[:, None] == lax.broadcasted_iota(jnp.int32, (1, NMOL_PAD),
                                                 1)).astype(jnp.float32)
  acc_ref[...] += lax.dot_general(onehot, hid, (((0,), (0,)), ((), ())),
                                  preferred_element_type=jnp.float32)
  cnt_ref[...] += lax.dot_general(onehot, jnp.ones_like(hid),
                                  (((0,), (0,)), ((), ())),
                                  preferred_element_type=jnp.float32)

  @pl.when(i == pl.num_programs(0) - 1)
  def _():
    out_ref[...] = acc_ref[...] / jnp.maximum(cnt_ref[...], 1.0)


def _k3(f_atoms, am, mol3, woaT, wohT, bo2):
  grid = (N_ATOMS // BA,)
  return pl.pallas_call(
      _k3_body,
      grid=grid,
      in_specs=[
          pl.BlockSpec((BA, H), lambda i: (i, 0)),
          pl.BlockSpec((BA, H), lambda i: (i, 0)),
          pl.BlockSpec((1, 1, BA), lambda i: (i, 0, 0)),
          pl.BlockSpec((H, H), lambda i: (0, 0)),
          pl.BlockSpec((H, H), lambda i: (0, 0)),
          pl.BlockSpec((1, H), lambda i: (0, 0)),
      ],
      out_specs=pl.BlockSpec((NMOL_PAD, H), lambda i: (0, 0)),
      out_shape=jax.ShapeDtypeStruct((NMOL_PAD, H), jnp.float32),
      scratch_shapes=[
          pltpu.VMEM((NMOL_PAD, H), jnp.float32),
          pltpu.VMEM((NMOL_PAD, H), jnp.float32),
      ],
  )(f_atoms, am, mol3, woaT, wohT, bo2)


def kernel(f_atoms, f_bonds, a2b, b2a, b2revb, mol_ids, W_i, W_h, W_o, b_o):
  a2b = a2b.astype(jnp.int32)
  b2a = b2a.astype(jnp.int32)
  b2revb = b2revb.astype(jnp.int32)
  mol_ids = mol_ids.astype(jnp.int32)

  wiT = W_i.T
  whT = W_h.T
  woaT = W_o[:, :H].T
  wohT = W_o[:, H:].T
  bo2 = b_o.reshape(1, H)

  n_extra = (N_PAD - N_ATOMS) * MAX_NB
  pad_idx = (jnp.arange(n_extra, dtype=jnp.int32) * 41) % N_BONDS
  idx_a2b = jnp.concatenate([a2b.reshape(-1), pad_idx])
  mol3 = mol_ids.reshape(N_ATOMS // BA, 1, BA)

  inp, mh1 = _k0(f_bonds.T, wiT, whT)
  amh1 = _segsum(mh1, idx_a2b)
  g1 = _combine(amh1, mh1, b2a, b2revb, inp, with_inp=False)
  mh2 = _k1(inp, g1, whT)
  amh2 = _segsum(mh2, idx_a2b)
  msg3 = _combine(amh2, mh2, b2a, b2revb, inp, with_inp=True)
  am3 = _segsum(msg3, idx_a2b)
  out = _k3(f_atoms, am3[:N_ATOMS], mol3, woaT, wohT, bo2)
  return out[:100]


# speedup vs baseline: 1.0271x; 1.0271x over previous
import functools

import jax
import jax.numpy as jnp
from jax import lax
from jax.experimental import pallas as pl
from jax.experimental.pallas import tpu as pltpu
from jax.experimental.pallas import tpu_sc as plsc

N_ATOMS = 10000
N_BONDS = 320000
MAX_NB = 32
H = 128
BOND_FDIM = 144

NC = 2
NS = 16
NW = NC * NS
APW = 320
N_PAD = NW * APW
BPW = N_BONDS // NW
GA = 1
NGROUPS = APW // GA
CB = 40
NCHUNK = BPW // CB
RING_C = 5

_mesh = plsc.VectorSubcoreMesh(
    core_axis_name="c", subcore_axis_name="s", num_cores=NC, num_subcores=NS)


def _wid():
  return lax.axis_index("c") * NS + lax.axis_index("s")


RING = 8


def _segsum_body(src_hbm, idx_hbm, out_hbm, idx_v, bufs, out_v, sems):
  base = _wid() * APW
  pltpu.sync_copy(idx_hbm.at[pl.ds(base * MAX_NB, APW * MAX_NB)], idx_v)

  def start(g, slot):
    pltpu.async_copy(
        src_hbm.at[idx_v.at[pl.ds(g * (GA * MAX_NB), GA * MAX_NB)]],
        bufs.at[slot], sems.at[slot])

  for s in range(RING):
    start(s, s)

  def outer(gg, carry):
    for slot in range(RING):
      g = RING * gg + slot
      buf = bufs.at[slot]
      pltpu.make_async_copy(
          src_hbm.at[idx_v.at[pl.ds(g * (GA * MAX_NB), GA * MAX_NB)]], buf,
          sems.at[slot]).wait()
      for a in range(GA):
        def rbody(kb, accs, a=a):
          r0 = a * MAX_NB + 4 * kb
          news = []
          for j in range(8):
            sl = pl.ds(16 * j, 16)
            v01 = buf[r0, sl] + buf[r0 + 1, sl]
            v23 = buf[r0 + 2, sl] + buf[r0 + 3, sl]
            news.append(accs[j] + (v01 + v23))
          return tuple(news)
        accs = lax.fori_loop(
            0, MAX_NB // 4, rbody,
            tuple(jnp.zeros((16,), jnp.float32) for _ in range(8)))
        r = slot * GA + a
        for j in range(8):
          out_v[r, 16 * j:16 * (j + 1)] = accs[j]

      @pl.when(g + RING < NGROUPS)
      def _():
        start(g + RING, slot)
    pltpu.sync_copy(out_v,
                    out_hbm.at[pl.ds(base + RING * GA * gg, RING * GA)])
    return carry

  lax.fori_loop(0, NGROUPS // RING, outer, 0)


def _segsum(src, idx_flat):
  fn = pl.kernel(
      _segsum_body,
      out_type=jax.ShapeDtypeStruct((N_PAD, H), jnp.float32),
      mesh=_mesh,
      scratch_types=[
          pltpu.VMEM((APW * MAX_NB,), jnp.int32),
          pltpu.VMEM((RING, GA * MAX_NB, H), jnp.float32),
          pltpu.VMEM((RING * GA, H), jnp.float32),
          pltpu.SemaphoreType.DMA((RING,)),
      ],
  )
  return fn(src, idx_flat)


def _combine_body(amh_hbm, mh_hbm, b2a_hbm, b2revb_hbm, inp_hbm, out_hbm,
                  idx1_v, idx2_v, bufs, out_bufs, sems, *, with_inp):
  base = _wid() * BPW
  pltpu.sync_copy(b2a_hbm.at[pl.ds(base, BPW)], idx1_v)
  pltpu.sync_copy(b2revb_hbm.at[pl.ds(base, BPW)], idx2_v)

  def start(c, slot):
    bufA, bufB, bufI = bufs.at[3 * slot], bufs.at[3 * slot + 1], bufs.at[
        3 * slot + 2]
    pltpu.async_copy(amh_hbm.at[idx1_v.at[pl.ds(c * CB, CB)]], bufA,
                     sems.at[3 * slot])
    pltpu.async_copy(mh_hbm.at[idx2_v.at[pl.ds(c * CB, CB)]], bufB,
                     sems.at[3 * slot + 1])
    if with_inp:
      pltpu.async_copy(inp_hbm.at[pl.ds(base + c * CB, CB)], bufI,
                       sems.at[3 * slot + 2])

  for s in range(RING_C):
    start(s, s)

  def outer(cc, carry):
    for slot in range(RING_C):
      c = RING_C * cc + slot
      bufA, bufB, bufI = bufs.at[3 * slot], bufs.at[3 * slot + 1], bufs.at[
          3 * slot + 2]
      out_v = out_bufs.at[slot]
      pltpu.make_async_copy(amh_hbm.at[idx1_v.at[pl.ds(c * CB, CB)]], bufA,
                            sems.at[3 * slot]).wait()
      pltpu.make_async_copy(mh_hbm.at[idx2_v.at[pl.ds(c * CB, CB)]], bufB,
                            sems.at[3 * slot + 1]).wait()
      if with_inp:
        pltpu.make_async_copy(inp_hbm.at[pl.ds(base + c * CB, CB)], bufI,
                              sems.at[3 * slot + 2]).wait()

      def rbody(rb, carry2):
        for rr in range(4):
          r = 4 * rb + rr
          for j in range(8):
            sl = pl.ds(16 * j, 16)
            v = bufA[r, sl] - bufB[r, sl]
            if with_inp:
              v = jnp.maximum(v + bufI[r, sl], 0.0)
            out_v[r, sl] = v
        return carry2

      lax.fori_loop(0, CB // 4, rbody, 0)
      pltpu.sync_copy(out_v, out_hbm.at[pl.ds(base + c * CB, CB)])

      @pl.when(c + RING_C < NCHUNK)
      def _():
        start(c + RING_C, slot)
    return carry

  lax.fori_loop(0, NCHUNK // RING_C, outer, 0)


def _combine(amh, mh, b2a, b2revb, inp, with_inp):
  fn = pl.kernel(
      functools.partial(_combine_body, with_inp=with_inp),
      out_type=jax.ShapeDtypeStruct((N_BONDS, H), jnp.float32),
      mesh=_mesh,
      scratch_types=[
          pltpu.VMEM((BPW,), jnp.int32),
          pltpu.VMEM((BPW,), jnp.int32),
          pltpu.VMEM((3 * RING_C, CB, H), jnp.float32),
          pltpu.VMEM((RING_C, CB, H), jnp.float32),
          pltpu.SemaphoreType.DMA((3 * RING_C,)),
      ],
  )
  return fn(amh, mh, b2a, b2revb, inp)


BE = 6400


def _k0_body(xT_ref, wiT_ref, whT_ref, inp_ref, mh_ref):
  inp = lax.dot_general(xT_ref[...], wiT_ref[...], (((0,), (0,)), ((), ())),
                        preferred_element_type=jnp.float32)
  inp_ref[...] = inp
  mh_ref[...] = jnp.dot(
      jnp.maximum(inp, 0.0), whT_ref[...], preferred_element_type=jnp.float32)


def _k0(f_bonds_T, wiT, whT):
  grid = (N_BONDS // BE,)
  return pl.pallas_call(
      _k0_body,
      grid=grid,
      in_specs=[
          pl.BlockSpec((BOND_FDIM, BE), lambda i: (0, i)),
          pl.BlockSpec((BOND_FDIM, H), lambda i: (0, 0)),
          pl.BlockSpec((H, H), lambda i: (0, 0)),
      ],
      out_specs=[
          pl.BlockSpec((BE, H), lambda i: (i, 0)),
          pl.BlockSpec((BE, H), lambda i: (i, 0)),
      ],
      out_shape=[
          jax.ShapeDtypeStruct((N_BONDS, H), jnp.float32),
          jax.ShapeDtypeStruct((N_BONDS, H), jnp.float32),
      ],
  )(f_bonds_T, wiT, whT)


def _k1_body(inp_ref, g_ref, whT_ref, mh_ref):
  m = jnp.maximum(inp_ref[...] + g_ref[...], 0.0)
  mh_ref[...] = jnp.dot(m, whT_ref[...], preferred_element_type=jnp.float32)


def _k1(inp, g, whT):
  grid = (N_BONDS // BE,)
  return pl.pallas_call(
      _k1_body,
      grid=grid,
      in_specs=[
          pl.BlockSpec((BE, H), lambda i: (i, 0)),
          pl.BlockSpec((BE, H), lambda i: (i, 0)),
          pl.BlockSpec((H, H), lambda i: (0, 0)),
      ],
      out_specs=pl.BlockSpec((BE, H), lambda i: (i, 0)),
      out_shape=jax.ShapeDtypeStruct((N_BONDS, H), jnp.float32),
  )(inp, g, whT)


BA = 2000
NMOL_PAD = 128


def _k3_body(fa_ref, am_ref, mol_ref, woaT_ref, wohT_ref, bo_ref, out_ref,
             acc_ref, cnt_ref):
  i = pl.program_id(0)

  @pl.when(i == 0)
  def _():
    acc_ref[...] = jnp.zeros_like(acc_ref)
    cnt_ref[...] = jnp.zeros_like(cnt_ref)

  hid = jnp.dot(fa_ref[...], woaT_ref[...], preferred_element_type=jnp.float32)
  hid = hid + jnp.dot(
      am_ref[...], wohT_ref[...], preferred_element_type=jnp.float32)
  hid = jnp.maximum(hid + bo_ref[...], 0.0)
  mol = mol_ref[0, 0, :]
  onehot = (mol[:, None] == lax.broadcasted_iota(jnp.int32, (1, NMOL_PAD),
                                                 1)).astype(jnp.float32)
  acc_ref[...] += lax.dot_general(onehot, hid, (((0,), (0,)), ((), ())),
                                  preferred_element_type=jnp.float32)
  cnt_ref[...] += lax.dot_general(onehot, jnp.ones_like(hid),
                                  (((0,), (0,)), ((), ())),
                                  preferred_element_type=jnp.float32)

  @pl.when(i == pl.num_programs(0) - 1)
  def _():
    out_ref[...] = acc_ref[...] / jnp.maximum(cnt_ref[...], 1.0)


def _k3(f_atoms, am, mol3, woaT, wohT, bo2):
  grid = (N_ATOMS // BA,)
  return pl.pallas_call(
      _k3_body,
      grid=grid,
      in_specs=[
          pl.BlockSpec((BA, H), lambda i: (i, 0)),
          pl.BlockSpec((BA, H), lambda i: (i, 0)),
          pl.BlockSpec((1, 1, BA), lambda i: (i, 0, 0)),
          pl.BlockSpec((H, H), lambda i: (0, 0)),
          pl.BlockSpec((H, H), lambda i: (0, 0)),
          pl.BlockSpec((1, H), lambda i: (0, 0)),
      ],
      out_specs=pl.BlockSpec((NMOL_PAD, H), lambda i: (0, 0)),
      out_shape=jax.ShapeDtypeStruct((NMOL_PAD, H), jnp.float32),
      scratch_shapes=[
          pltpu.VMEM((NMOL_PAD, H), jnp.float32),
          pltpu.VMEM((NMOL_PAD, H), jnp.float32),
      ],
  )(f_atoms, am, mol3, woaT, wohT, bo2)


def kernel(f_atoms, f_bonds, a2b, b2a, b2revb, mol_ids, W_i, W_h, W_o, b_o):
  a2b = a2b.astype(jnp.int32)
  b2a = b2a.astype(jnp.int32)
  b2revb = b2revb.astype(jnp.int32)
  mol_ids = mol_ids.astype(jnp.int32)

  wiT = W_i.T
  whT = W_h.T
  woaT = W_o[:, :H].T
  wohT = W_o[:, H:].T
  bo2 = b_o.reshape(1, H)

  n_extra = (N_PAD - N_ATOMS) * MAX_NB
  pad_idx = (jnp.arange(n_extra, dtype=jnp.int32) * 41) % N_BONDS
  idx_a2b = jnp.concatenate([a2b.reshape(-1), pad_idx])
  mol3 = mol_ids.reshape(N_ATOMS // BA, 1, BA)

  inp, mh1 = _k0(f_bonds.T, wiT, whT)
  amh1 = _segsum(mh1, idx_a2b)
  g1 = _combine(amh1, mh1, b2a, b2revb, inp, with_inp=False)
  mh2 = _k1(inp, g1, whT)
  amh2 = _segsum(mh2, idx_a2b)
  msg3 = _combine(amh2, mh2, b2a, b2revb, inp, with_inp=True)
  am3 = _segsum(msg3, idx_a2b)
  out = _k3(f_atoms, am3[:N_ATOMS], mol3, woaT, wohT, bo2)
  return out[:100]
